# Initial kernel scaffold; baseline (speedup 1.0000x reference)
import jax
import jax.numpy as jnp
from jax.experimental import pallas as pl


def kernel(features, type_mask, edge_metapath_indices, dst, W_ih, W_hh, b_ih, b_hh, attn1_w, attn2):
    raise NotImplementedError("write your pallas kernel here")


# speedup vs baseline: 15.5685x; 15.5685x over previous
import functools

import jax
import jax.numpy as jnp
from jax import lax
from jax.experimental import pallas as pl
from jax.experimental.pallas import tpu as pltpu
from jax.experimental.pallas import tpu_sc as plsc

N = 10000
E = 160000
L = 3
D = 128
NH = 2
H = NH * D
G4 = 4 * H
ALPHA = 0.01

NC = 2
NS = 16
NW = NC * NS

PW = (L * E) // NW
CG = 120
NCH = PW // CG


def _sc_gather(features, idx_flat):
    mesh = plsc.VectorSubcoreMesh(
        core_axis_name="c", subcore_axis_name="s", num_cores=NC,
        num_subcores=NS)

    @functools.partial(
        pl.kernel,
        out_type=jax.ShapeDtypeStruct((L * E, D), jnp.float32),
        mesh=mesh,
        scratch_types=[
            pltpu.VMEM((CG,), jnp.int32),
            pltpu.VMEM((CG, D), jnp.float32),
            pltpu.SemaphoreType.DMA,
        ],
    )
    def gather_k(feat_hbm, idx_hbm, out_hbm, idx_v, rows_v, sem):
        wid = lax.axis_index("s") * NC + lax.axis_index("c")

        def body(j, carry):
            base = wid * PW + j * CG
            pltpu.sync_copy(idx_hbm.at[pl.ds(base, CG)], idx_v)
            pltpu.async_copy(feat_hbm.at[idx_v], rows_v, sem).wait()
            pltpu.sync_copy(rows_v, out_hbm.at[pl.ds(base, CG)])
            return carry

        lax.fori_loop(0, NCH, body, 0)

    return gather_k(features, idx_flat)


BE = 640
NB = E // BE


def _lstm_body(x0r, x1r, x2r, wihr, whhr, br, a1r, a2r, w_out, e_out):
    x0 = x0r[0]
    x1 = x1r[0]
    x2 = x2r[0]
    wih = wihr[...]
    whh = whhr[...]
    b = br[...]

    def step(x, h, c, first):
        g = jnp.dot(x, wih, preferred_element_type=jnp.float32) + b
        if not first:
            g = g + jnp.dot(h, whh, preferred_element_type=jnp.float32)
        i = jax.nn.sigmoid(g[:, 0:H])
        f = jax.nn.sigmoid(g[:, H:2 * H])
        gg = jnp.tanh(g[:, 2 * H:3 * H])
        o = jax.nn.sigmoid(g[:, 3 * H:4 * H])
        c = (i * gg) if first else (f * c + i * gg)
        h = o * jnp.tanh(c)
        return h, c

    h, c = step(x0, None, None, True)
    h, c = step(x1, h, c, False)
    h, c = step(x2, h, c, False)

    h0 = h[:, 0:D]
    h1 = h[:, D:2 * D]
    a1_0 = jnp.sum(x2 * a1r[0:1, :], axis=1, keepdims=True)
    a1_1 = jnp.sum(x2 * a1r[1:2, :], axis=1, keepdims=True)
    a2_0 = jnp.sum(h0 * a2r[0:1, :], axis=1, keepdims=True)
    a2_1 = jnp.sum(h1 * a2r[1:2, :], axis=1, keepdims=True)
    a0 = a1_0 + a2_0
    a1_ = a1_1 + a2_1
    a0 = jnp.where(a0 >= 0, a0, ALPHA * a0)
    a1_ = jnp.where(a1_ >= 0, a1_, ALPHA * a1_)
    e0 = jnp.exp(a0)
    e1 = jnp.exp(a1_)
    w_out[...] = jnp.concatenate([h0 * e0, h1 * e1], axis=1)
    z = jnp.zeros((x0.shape[0], D - 2), dtype=jnp.float32)
    e_out[...] = jnp.concatenate([e0, e1, z], axis=1)


def _tc_encode(edata, W_ihT, W_hhT, bias, attn1, attn2):
    return pl.pallas_call(
        _lstm_body,
        grid=(NB,),
        in_specs=[
            pl.BlockSpec((1, BE, D), lambda i: (0, i, 0)),
            pl.BlockSpec((1, BE, D), lambda i: (1, i, 0)),
            pl.BlockSpec((1, BE, D), lambda i: (2, i, 0)),
            pl.BlockSpec((D, G4), lambda i: (0, 0)),
            pl.BlockSpec((H, G4), lambda i: (0, 0)),
            pl.BlockSpec((1, G4), lambda i: (0, 0)),
            pl.BlockSpec((NH, D), lambda i: (0, 0)),
            pl.BlockSpec((NH, D), lambda i: (0, 0)),
        ],
        out_specs=[
            pl.BlockSpec((BE, H), lambda i: (i, 0)),
            pl.BlockSpec((BE, D), lambda i: (i, 0)),
        ],
        out_shape=[
            jax.ShapeDtypeStruct((E, H), jnp.float32),
            jax.ShapeDtypeStruct((E, D), jnp.float32),
        ],
        compiler_params=pltpu.CompilerParams(
            dimension_semantics=("arbitrary",)),
    )(edata, edata, edata, W_ihT, W_hhT, bias, attn1, attn2)


NPAD = 10240
PT = NPAD // NS
CE = 80
ET = E // NS
ECH = ET // CE
ZCH = PT // CE


def _sc_scatter(w, dst, zrows):
    mesh = plsc.VectorSubcoreMesh(
        core_axis_name="c", subcore_axis_name="s", num_cores=NC,
        num_subcores=NS)

    @functools.partial(
        pl.kernel,
        out_type=jax.ShapeDtypeStruct((NH * NPAD, D), jnp.float32),
        mesh=mesh,
        scratch_types=[
            pltpu.VMEM((CE,), jnp.int32),
            pltpu.VMEM((CE, D), jnp.float32),
            pltpu.VMEM_SHARED((NPAD, D), jnp.float32),
        ],
    )
    def scatter_k(w_hbm, dst_hbm, z_hbm, acc_out, idx_v, rows_v, acc_sh):
        c = lax.axis_index("c")
        s = lax.axis_index("s")

        pltpu.sync_copy(z_hbm, rows_v)
        for k in range(ZCH):
            r0 = s * PT + k * CE
            pltpu.sync_copy(rows_v, acc_sh.at[pl.ds(r0, CE)])
        plsc.subcore_barrier()

        def body(j, carry):
            base = s * ET + j * CE
            pltpu.sync_copy(dst_hbm.at[pl.ds(base, CE)], idx_v)
            pltpu.sync_copy(w_hbm.at[pl.ds(base, CE), pl.ds(c * D, D)],
                            rows_v)
            pltpu.sync_copy(rows_v, acc_sh.at[idx_v], add=True)
            return carry

        lax.fori_loop(0, ECH, body, 0)
        plsc.subcore_barrier()

        for k in range(ZCH):
            r0 = s * PT + k * CE
            pltpu.sync_copy(acc_sh.at[pl.ds(r0, CE)], rows_v)
            pltpu.sync_copy(rows_v, acc_out.at[pl.ds(c * NPAD + r0, CE)])

    return scatter_k(w, dst, zrows).reshape(NH, NPAD, D)


CE2 = 40
ET2 = E // NW
ECH2 = ET2 // CE2
ZCH2 = PT // CE2


def _sc_scatter_e(e128, dst, zrows2):
    mesh = plsc.VectorSubcoreMesh(
        core_axis_name="c", subcore_axis_name="s", num_cores=NC,
        num_subcores=NS)

    @functools.partial(
        pl.kernel,
        out_type=jax.ShapeDtypeStruct((NC * NPAD, D), jnp.float32),
        mesh=mesh,
        scratch_types=[
            pltpu.VMEM((CE2,), jnp.int32),
            pltpu.VMEM((CE2, D), jnp.float32),
            pltpu.VMEM_SHARED((NPAD, D), jnp.float32),
        ],
    )
    def scatter_e_k(e_hbm, dst_hbm, z_hbm, ssum_out, idx_v, e_v, ssum_sh):
        c = lax.axis_index("c")
        s = lax.axis_index("s")

        pltpu.sync_copy(z_hbm, e_v)
        for k in range(ZCH2):
            r0 = s * PT + k * CE2
            pltpu.sync_copy(e_v, ssum_sh.at[pl.ds(r0, CE2)])
        plsc.subcore_barrier()

        def body(j, carry):
            base = (c * NS + s) * ET2 + j * CE2
            pltpu.sync_copy(dst_hbm.at[pl.ds(base, CE2)], idx_v)
            pltpu.sync_copy(e_hbm.at[pl.ds(base, CE2)], e_v)
            pltpu.sync_copy(e_v, ssum_sh.at[idx_v], add=True)
            return carry

        lax.fori_loop(0, ECH2, body, 0)
        plsc.subcore_barrier()

        for k in range(ZCH2):
            r0 = s * PT + k * CE2
            pltpu.sync_copy(ssum_sh.at[pl.ds(r0, CE2)], e_v)
            pltpu.sync_copy(e_v, ssum_out.at[pl.ds(c * NPAD + r0, CE2)])

    return scatter_e_k(e128, dst, zrows2).reshape(NC, NPAD, D)


BN = 640
NBN = NPAD // BN


def _div_body(accr, ssumr, outr):
    a0 = accr[0]
    a1 = accr[1]
    st = ssumr[0] + ssumr[1]
    s0 = st[:, 0:1]
    s1 = st[:, 1:2]
    s0 = jnp.where(s0 != 0, s0, 1.0)
    s1 = jnp.where(s1 != 0, s1, 1.0)
    outr[...] = jnp.concatenate([a0 / s0, a1 / s1], axis=1)


def _tc_divide(acc, ssum):
    return pl.pallas_call(
        _div_body,
        grid=(NBN,),
        in_specs=[
            pl.BlockSpec((NH, BN, D), lambda i: (0, i, 0)),
            pl.BlockSpec((NC, BN, D), lambda i: (0, i, 0)),
        ],
        out_specs=pl.BlockSpec((BN, H), lambda i: (i, 0)),
        out_shape=jax.ShapeDtypeStruct((NPAD, H), jnp.float32),
        compiler_params=pltpu.CompilerParams(
            dimension_semantics=("arbitrary",)),
    )(acc, ssum)


def kernel(features, type_mask, edge_metapath_indices, dst, W_ih, W_hh,
           b_ih, b_hh, attn1_w, attn2):
    del type_mask
    idx_flat = jnp.transpose(edge_metapath_indices).reshape(L * E)
    edata = _sc_gather(features, idx_flat).reshape(L, E, D)

    W_ihT = jnp.transpose(W_ih)
    W_hhT = jnp.transpose(W_hh)
    bias = (b_ih + b_hh).reshape(1, G4)
    attn2f = attn2.reshape(NH, D)
    w, e128 = _tc_encode(edata, W_ihT, W_hhT, bias, attn1_w, attn2f)

    zrows = jnp.zeros((CE, D), dtype=jnp.float32)
    acc = _sc_scatter(w, dst, zrows)
    zrows2 = jnp.zeros((CE2, D), dtype=jnp.float32)
    ssum = _sc_scatter_e(e128, dst, zrows2)

    out = _tc_divide(acc, ssum)
    return out[:N].reshape(N, NH, D)
